# baseline (speedup 1.0000x reference)
import jax
import jax.numpy as jnp
from jax.experimental import pallas as pl
from jax.experimental.pallas import tpu as pltpu

ALPHA = 1.702
LIMIT = 7.0

_TT = 512


def _body(x_ref, wg_ref, wu_ref, bg_ref, bu_ref, wd_ref, bd_ref, rw_ref, o_ref):
    e = pl.program_id(1)
    x = x_ref[...]
    gate = jnp.dot(x, wg_ref[0], preferred_element_type=jnp.float32) + bg_ref[0]
    up = jnp.dot(x, wu_ref[0], preferred_element_type=jnp.float32) + bu_ref[0]
    gate = jnp.minimum(gate, LIMIT)
    up = jnp.clip(up, -LIMIT, LIMIT)
    glu = gate * jax.nn.sigmoid(gate * ALPHA)
    gated = (up + 1.0) * glu
    out = jnp.dot(gated, wd_ref[0], preferred_element_type=jnp.float32) + bd_ref[0]
    contrib = out * rw_ref[0]

    @pl.when(e == 0)
    def _():
        o_ref[...] = contrib

    @pl.when(e != 0)
    def _():
        o_ref[...] += contrib


def kernel(hidden_states, router_indices, routing_weights, gate_up_proj,
           gate_up_proj_bias, down_proj, down_proj_bias):
    del router_indices
    T, H = hidden_states.shape
    E, _, D2 = gate_up_proj.shape
    D = D2 // 2

    wg = gate_up_proj[:, :, 0::2]
    wu = gate_up_proj[:, :, 1::2]
    bg = gate_up_proj_bias[:, None, 0::2]
    bu = gate_up_proj_bias[:, None, 1::2]
    bd = down_proj_bias[:, None, :]
    rw = jnp.transpose(routing_weights)[:, :, None]

    num_t = T // _TT
    grid = (num_t, E)

    return pl.pallas_call(
        _body,
        grid=grid,
        in_specs=[
            pl.BlockSpec((_TT, H), lambda t, e: (t, 0)),
            pl.BlockSpec((1, H, D), lambda t, e: (e, 0, 0)),
            pl.BlockSpec((1, H, D), lambda t, e: (e, 0, 0)),
            pl.BlockSpec((1, 1, D), lambda t, e: (e, 0, 0)),
            pl.BlockSpec((1, 1, D), lambda t, e: (e, 0, 0)),
            pl.BlockSpec((1, D, H), lambda t, e: (e, 0, 0)),
            pl.BlockSpec((1, 1, H), lambda t, e: (e, 0, 0)),
            pl.BlockSpec((1, _TT, 1), lambda t, e: (e, t, 0)),
        ],
        out_specs=pl.BlockSpec((_TT, H), lambda t, e: (t, 0)),
        out_shape=jax.ShapeDtypeStruct((T, H), jnp.float32),
        compiler_params=pltpu.CompilerParams(
            dimension_semantics=("arbitrary", "arbitrary"),
        ),
    )(hidden_states, wg, wu, bg, bu, down_proj, bd, rw)


# speedup vs baseline: 7.6329x; 7.6329x over previous
import jax
import jax.numpy as jnp
from jax.experimental import pallas as pl
from jax.experimental.pallas import tpu as pltpu

ALPHA = 1.702
LIMIT = 7.0

_TT = 512


def _body(x_ref, wg_ref, wu_ref, bg_ref, bu_ref, wd_ref, bd_ref, rw_ref, o_ref):
    e = pl.program_id(1)
    x = x_ref[...]
    gate = jnp.dot(x, wg_ref[0], preferred_element_type=jnp.float32) + bg_ref[0]
    up = jnp.dot(x, wu_ref[0], preferred_element_type=jnp.float32) + bu_ref[0]
    gate = jnp.minimum(gate, LIMIT)
    up = jnp.clip(up, -LIMIT, LIMIT)
    glu = gate * jax.nn.sigmoid(gate * ALPHA)
    gated = (up + 1.0) * glu
    out = jnp.dot(gated, wd_ref[0], preferred_element_type=jnp.float32) + bd_ref[0]
    contrib = out * rw_ref[0]

    @pl.when(e == 0)
    def _():
        o_ref[...] = contrib

    @pl.when(e != 0)
    def _():
        o_ref[...] += contrib


def kernel(hidden_states, router_indices, routing_weights, gate_up_proj,
           gate_up_proj_bias, down_proj, down_proj_bias):
    del router_indices
    T, H = hidden_states.shape
    E, _, D2 = gate_up_proj.shape
    D = D2 // 2

    wgu = jnp.transpose(gate_up_proj.reshape(E, H, D, 2), (3, 0, 1, 2))
    wg = wgu[0]
    wu = wgu[1]
    bg = gate_up_proj_bias[:, None, 0::2]
    bu = gate_up_proj_bias[:, None, 1::2]
    bd = down_proj_bias[:, None, :]
    rw = jnp.transpose(routing_weights)[:, :, None]

    num_t = T // _TT
    grid = (num_t, E)

    return pl.pallas_call(
        _body,
        grid=grid,
        in_specs=[
            pl.BlockSpec((_TT, H), lambda t, e: (t, 0)),
            pl.BlockSpec((1, H, D), lambda t, e: (e, 0, 0)),
            pl.BlockSpec((1, H, D), lambda t, e: (e, 0, 0)),
            pl.BlockSpec((1, 1, D), lambda t, e: (e, 0, 0)),
            pl.BlockSpec((1, 1, D), lambda t, e: (e, 0, 0)),
            pl.BlockSpec((1, D, H), lambda t, e: (e, 0, 0)),
            pl.BlockSpec((1, 1, H), lambda t, e: (e, 0, 0)),
            pl.BlockSpec((1, _TT, 1), lambda t, e: (e, t, 0)),
        ],
        out_specs=pl.BlockSpec((_TT, H), lambda t, e: (t, 0)),
        out_shape=jax.ShapeDtypeStruct((T, H), jnp.float32),
        compiler_params=pltpu.CompilerParams(
            dimension_semantics=("arbitrary", "arbitrary"),
        ),
    )(hidden_states, wg, wu, bg, bu, down_proj, bd, rw)


# speedup vs baseline: 8.4678x; 1.1094x over previous
import jax
import jax.numpy as jnp
from jax.experimental import pallas as pl
from jax.experimental.pallas import tpu as pltpu

ALPHA = 1.702
LIMIT = 7.0

_TT = 512


def _body(x_ref, wg_ref, wu_ref, bg_ref, bu_ref, wd_ref, bd_ref, rw_ref, o_ref):
    e = pl.program_id(1)
    x = x_ref[...]
    gate = jnp.dot(x, wg_ref[0], preferred_element_type=jnp.float32) + bg_ref[0]
    up = jnp.dot(x, wu_ref[0], preferred_element_type=jnp.float32) + bu_ref[0]
    gate = jnp.minimum(gate, LIMIT)
    up = jnp.clip(up, -LIMIT, LIMIT)
    glu = gate * jax.nn.sigmoid(gate * ALPHA)
    gated = ((up + 1.0) * glu).astype(jnp.bfloat16)
    out = jnp.dot(gated, wd_ref[0], preferred_element_type=jnp.float32) + bd_ref[0]
    contrib = out * rw_ref[0]

    @pl.when(e == 0)
    def _():
        o_ref[...] = contrib

    @pl.when(e != 0)
    def _():
        o_ref[...] += contrib


def kernel(hidden_states, router_indices, routing_weights, gate_up_proj,
           gate_up_proj_bias, down_proj, down_proj_bias):
    del router_indices
    T, H = hidden_states.shape
    E, _, D2 = gate_up_proj.shape
    D = D2 // 2

    x16 = hidden_states.astype(jnp.bfloat16)
    wgu = jnp.transpose(gate_up_proj.astype(jnp.bfloat16).reshape(E, H, D, 2),
                        (3, 0, 1, 2))
    wg = wgu[0]
    wu = wgu[1]
    wd = down_proj.astype(jnp.bfloat16)
    bg = gate_up_proj_bias[:, None, 0::2]
    bu = gate_up_proj_bias[:, None, 1::2]
    bd = down_proj_bias[:, None, :]
    rw = jnp.transpose(routing_weights)[:, :, None]

    num_t = T // _TT
    grid = (num_t, E)

    return pl.pallas_call(
        _body,
        grid=grid,
        in_specs=[
            pl.BlockSpec((_TT, H), lambda t, e: (t, 0)),
            pl.BlockSpec((1, H, D), lambda t, e: (e, 0, 0)),
            pl.BlockSpec((1, H, D), lambda t, e: (e, 0, 0)),
            pl.BlockSpec((1, 1, D), lambda t, e: (e, 0, 0)),
            pl.BlockSpec((1, 1, D), lambda t, e: (e, 0, 0)),
            pl.BlockSpec((1, D, H), lambda t, e: (e, 0, 0)),
            pl.BlockSpec((1, 1, H), lambda t, e: (e, 0, 0)),
            pl.BlockSpec((1, _TT, 1), lambda t, e: (e, t, 0)),
        ],
        out_specs=pl.BlockSpec((_TT, H), lambda t, e: (t, 0)),
        out_shape=jax.ShapeDtypeStruct((T, H), jnp.float32),
        compiler_params=pltpu.CompilerParams(
            dimension_semantics=("arbitrary", "arbitrary"),
        ),
    )(x16, wg, wu, bg, bu, wd, bd, rw)
